# baseline (speedup 1.0000x reference)
import functools

import jax
import jax.numpy as jnp
from jax import lax
from jax.experimental import pallas as pl
from jax.experimental.pallas import tpu as pltpu
from jax.experimental.pallas import tpu_sc as plsc

B = 1024
C = 3
H = 224
W_DIM = 224
HW = H * W_DIM
E = 64

NTEC = 32
HS = 32
H0 = H - HS

BH = 16
NH = H0 // BH

WCH = 56
NCH = W_DIM // WCH
NCHUNKS = C * NCH


def _tc_pool_kernel(x_ref, out_ref):
    j = pl.program_id(1)
    s = jnp.sum(x_ref[0], axis=(0, 1), keepdims=True)[0]

    @pl.when(j == 0)
    def _():
        out_ref[0] = s

    @pl.when(j != 0)
    def _():
        out_ref[0] += s


_tc_pool = pl.pallas_call(
    _tc_pool_kernel,
    grid=(C, NH),
    in_specs=[pl.BlockSpec((1, BH, W_DIM, B), lambda c, j: (c, j, 0, 0))],
    out_specs=pl.BlockSpec((1, 1, B), lambda c, j: (c, 0, 0)),
    out_shape=jax.ShapeDtypeStruct((C, 1, B), jnp.float32),
)


_sc_mesh = plsc.VectorSubcoreMesh(core_axis_name="c", subcore_axis_name="s")


@functools.partial(
    pl.kernel,
    mesh=_sc_mesh,
    out_type=jax.ShapeDtypeStruct((C, NTEC, B), jnp.float32),
    scratch_types=[
        pltpu.VMEM((WCH, B), jnp.float32),
        pltpu.VMEM((WCH, B), jnp.float32),
        pltpu.VMEM((B,), jnp.float32),
        pltpu.SemaphoreType.DMA,
        pltpu.SemaphoreType.DMA,
    ],
)
def _sc_pool(x_hbm, out_hbm, buf0, buf1, acc, sem0, sem1):
    w = lax.axis_index("s") * 2 + lax.axis_index("c")
    h = H0 + w

    def start(ci, buf, sem):
        c = ci // NCH
        k = ci % NCH
        pltpu.async_copy(x_hbm.at[c, h, pl.ds(k * WCH, WCH), :], buf, sem)

    def zero_acc():
        z = jnp.zeros((16,), jnp.float32)
        for j in range(B // 16):
            acc[pl.ds(j * 16, 16)] = z

    def accumulate(buf):
        def body(j, carry):
            v = buf[0, pl.ds(j * 16, 16)]
            for row in range(1, WCH):
                v = v + buf[row, pl.ds(j * 16, 16)]
            plsc.addupdate(acc.at[pl.ds(j * 16, 16)], v)
            return carry
        lax.fori_loop(0, B // 16, body, 0)

    def drain(ci):
        c = ci // NCH
        pltpu.sync_copy(acc, out_hbm.at[c, w, :])

    zero_acc()
    start(0, buf0, sem0)
    start(1, buf1, sem1)

    def loop_body(i, carry):
        ci0 = i * 2
        ci1 = i * 2 + 1

        pltpu.make_async_copy(x_hbm.at[0, 0, pl.ds(0, WCH), :], buf0,
                              sem0).wait()
        accumulate(buf0)

        @pl.when(ci0 % NCH == NCH - 1)
        def _():
            drain(ci0)
            zero_acc()

        @pl.when(ci0 + 2 < NCHUNKS)
        def _():
            start(ci0 + 2, buf0, sem0)

        pltpu.make_async_copy(x_hbm.at[0, 0, pl.ds(0, WCH), :], buf1,
                              sem1).wait()
        accumulate(buf1)

        @pl.when(ci1 % NCH == NCH - 1)
        def _():
            drain(ci1)
            zero_acc()

        @pl.when(ci1 + 2 < NCHUNKS)
        def _():
            start(ci1 + 2, buf1, sem1)

        return carry

    lax.fori_loop(0, NCHUNKS // 2, loop_body, 0)


def _routing_kernel(tcs_ref, scp_ref, w_ref, b_ref, gates_ref, idx_ref):
    n = jnp.float32(HW)
    sums = [tcs_ref[c] + jnp.sum(scp_ref[c], axis=0, keepdims=True)
            for c in range(C)]
    p0 = (sums[0] / n).astype(jnp.bfloat16).astype(jnp.float32)
    p1 = (sums[1] / n).astype(jnp.bfloat16).astype(jnp.float32)
    p2 = (sums[2] / n).astype(jnp.bfloat16).astype(jnp.float32)
    wb = w_ref[...].astype(jnp.bfloat16).astype(jnp.float32)
    logits = (wb[:, 0:1] * p0 + wb[:, 1:2] * p1) + wb[:, 2:3] * p2
    logits = logits + b_ref[...]

    iota = jax.lax.broadcasted_iota(jnp.int32, (E, B), 0)
    m0 = jnp.max(logits, axis=0, keepdims=True)
    idx0 = jnp.min(jnp.where(logits == m0, iota, E), axis=0, keepdims=True)
    masked = jnp.where(iota == idx0, jnp.finfo(jnp.float32).min, logits)
    m1 = jnp.max(masked, axis=0, keepdims=True)
    idx1 = jnp.min(jnp.where(masked == m1, iota, E), axis=0, keepdims=True)

    e1 = jnp.exp(m1 - m0)
    denom = 1.0 + e1
    g0 = 1.0 / denom
    g1 = e1 / denom

    gates_ref[...] = jnp.where(iota == idx0, g0,
                               jnp.where(iota == idx1, g1, 0.0))
    idx_ref[...] = jnp.concatenate([idx0, idx1], axis=0)


_route = pl.pallas_call(
    _routing_kernel,
    in_specs=[
        pl.BlockSpec((C, 1, B), lambda: (0, 0, 0)),
        pl.BlockSpec((C, NTEC, B), lambda: (0, 0, 0)),
        pl.BlockSpec((E, C), lambda: (0, 0)),
        pl.BlockSpec((E, 1), lambda: (0, 0)),
    ],
    out_specs=[
        pl.BlockSpec((E, B), lambda: (0, 0)),
        pl.BlockSpec((2, B), lambda: (0, 0)),
    ],
    out_shape=[
        jax.ShapeDtypeStruct((E, B), jnp.float32),
        jax.ShapeDtypeStruct((2, B), jnp.int32),
    ],
)


def kernel(x, W, b):
    xt = jnp.transpose(x, (1, 2, 3, 0))
    scp = _sc_pool(xt)
    tcs = _tc_pool(xt)
    gates_t, idx_t = _route(tcs, scp, W, b.reshape(E, 1))
    return (gates_t.T, idx_t.T)


# speedup vs baseline: 1.0776x; 1.0776x over previous
import jax
import jax.numpy as jnp
from jax.experimental import pallas as pl
from jax.experimental.pallas import tpu as pltpu

B = 1024
C = 3
H = 224
W_DIM = 224
HW = H * W_DIM
E = 64

BH = 16
NH = H // BH


def _router_kernel(x_ref, w_ref, b_ref, gates_ref, idx_ref, acc_ref):
    c = pl.program_id(0)
    j = pl.program_id(1)
    s = jnp.sum(x_ref[0], axis=(0, 1), keepdims=True)[0]

    @pl.when(j == 0)
    def _():
        acc_ref[c] = s

    @pl.when(j != 0)
    def _():
        acc_ref[c] += s

    @pl.when((c == C - 1) & (j == NH - 1))
    def _finish():
        n = jnp.float32(HW)
        p0 = (acc_ref[0] / n).astype(jnp.bfloat16).astype(jnp.float32)
        p1 = (acc_ref[1] / n).astype(jnp.bfloat16).astype(jnp.float32)
        p2 = (acc_ref[2] / n).astype(jnp.bfloat16).astype(jnp.float32)
        wb = w_ref[...].astype(jnp.bfloat16).astype(jnp.float32)
        logits = (wb[:, 0:1] * p0 + wb[:, 1:2] * p1) + wb[:, 2:3] * p2
        logits = logits + b_ref[...]

        iota = jax.lax.broadcasted_iota(jnp.int32, (E, B), 0)
        m0 = jnp.max(logits, axis=0, keepdims=True)
        idx0 = jnp.min(jnp.where(logits == m0, iota, E), axis=0,
                       keepdims=True)
        masked = jnp.where(iota == idx0, jnp.finfo(jnp.float32).min, logits)
        m1 = jnp.max(masked, axis=0, keepdims=True)
        idx1 = jnp.min(jnp.where(masked == m1, iota, E), axis=0,
                       keepdims=True)

        e1 = jnp.exp(m1 - m0)
        denom = 1.0 + e1
        g0 = 1.0 / denom
        g1 = e1 / denom

        gates_ref[...] = jnp.where(iota == idx0, g0,
                                   jnp.where(iota == idx1, g1, 0.0))
        idx_ref[...] = jnp.concatenate([idx0, idx1], axis=0)


_router = pl.pallas_call(
    _router_kernel,
    grid=(C, NH),
    in_specs=[
        pl.BlockSpec((1, BH, W_DIM, B), lambda c, j: (c, j, 0, 0)),
        pl.BlockSpec((E, C), lambda c, j: (0, 0)),
        pl.BlockSpec((E, 1), lambda c, j: (0, 0)),
    ],
    out_specs=[
        pl.BlockSpec((E, B), lambda c, j: (0, 0)),
        pl.BlockSpec((2, B), lambda c, j: (0, 0)),
    ],
    out_shape=[
        jax.ShapeDtypeStruct((E, B), jnp.float32),
        jax.ShapeDtypeStruct((2, B), jnp.int32),
    ],
    scratch_shapes=[pltpu.VMEM((C, 1, B), jnp.float32)],
)


def kernel(x, W, b):
    xt = jnp.transpose(x, (1, 2, 3, 0))
    gates_t, idx_t = _router(xt, W, b.reshape(E, 1))
    return (gates_t.T, idx_t.T)
